# Initial kernel scaffold; baseline (speedup 1.0000x reference)
import functools

import jax
import jax.numpy as jnp
from jax import lax
from jax.experimental import pallas as pl
from jax.experimental.pallas import tpu as pltpu
from jax.experimental.pallas import tpu_sc as plsc

_N = 10000
_E = 160000
_D = 128
_NZ = 10
_NC = 10
_V = 1.0

_BLK = 1000
_GRID = _N // _BLK

_TILES = 16
_K = 128
_E_PAD = ((_E + _TILES * _K - 1) // (_TILES * _K)) * (_TILES * _K)
_EPT = _E_PAD // _TILES
_NB = _EPT // _K
_RPT = _N // _TILES

_HI = lax.Precision.HIGHEST


def _dot(a, b):
    return jnp.dot(a, b, precision=_HI, preferred_element_type=jnp.float32)


def _relu(a):
    return jnp.maximum(a, 0.0)


def _leaky(a):
    return jnp.where(a > 0, a, 0.01 * a)


def _softmax(a):
    m = jnp.max(a, axis=1, keepdims=True)
    e = jnp.exp(a - m)
    return e / jnp.sum(e, axis=1, keepdims=True)


def _norm2(a):
    n = jnp.sqrt(jnp.sum(a * a, axis=1, keepdims=True))
    return a / jnp.maximum(n, 1e-12)


def _full(shape):
    return pl.BlockSpec(shape, lambda i: (0,) * len(shape))


def _rows(shape):
    return pl.BlockSpec(shape, lambda i: (i,) + (0,) * (len(shape) - 1))


def _t1_body(x_ref, w1, b1, w2, b2, w3, b3, wz, bz, wd1, bd1, wd2, bd2,
             wd3, bd3, wx, bx, g0p, clusT,
             h1_o, h2_o, h3_o, z_o, q_o, xbar_o, s1_o):
    x = x_ref[...]
    h1 = _relu(_dot(x, w1[...]) + b1[...])
    h2 = _relu(_dot(h1, w2[...]) + b2[...])
    h3 = _relu(_dot(h2, w3[...]) + b3[...])
    z = _dot(h3, wz[...]) + bz[...]
    d1 = _relu(_dot(z, wd1[...]) + bd1[...])
    d2 = _relu(_dot(d1, wd2[...]) + bd2[...])
    d3 = _relu(_dot(d2, wd3[...]) + bd3[...])
    xbar_o[...] = _dot(d3, wx[...]) + bx[...]

    ct = clusT[...]
    z2s = jnp.sum(z * z, axis=1, keepdims=True)
    c2s = jnp.sum(ct * ct, axis=0)[None, :]
    dist = z2s + c2s - 2.0 * _dot(z, ct)
    q = 1.0 / (1.0 + dist / _V)
    q_o[...] = q / jnp.sum(q, axis=1, keepdims=True)

    s1 = _dot(x, g0p[...])
    for c in range(4):
        s1_o[c] = s1[:, c * 128:(c + 1) * 128]

    h1_o[...] = h1
    h2_o[...] = h2
    h3_o[...] = h3
    z_o[...] = z


def _stage_t1(x, p, clusT, g0p):
    outs = [
        jax.ShapeDtypeStruct((_N, 500), jnp.float32),
        jax.ShapeDtypeStruct((_N, 500), jnp.float32),
        jax.ShapeDtypeStruct((_N, 2000), jnp.float32),
        jax.ShapeDtypeStruct((_N, _NZ), jnp.float32),
        jax.ShapeDtypeStruct((_N, _NC), jnp.float32),
        jax.ShapeDtypeStruct((_N, _D), jnp.float32),
        jax.ShapeDtypeStruct((4, _N, 128), jnp.float32),
    ]
    args = [x]
    in_specs = [_rows((_BLK, _D))]
    for name in ('enc1', 'enc2', 'enc3', 'zl', 'dec1', 'dec2', 'dec3', 'xbar'):
        w, b = p[name]
        args += [w, b.reshape(1, -1)]
        in_specs += [_full(w.shape), _full((1, b.shape[0]))]
    args += [g0p, clusT]
    in_specs += [_full(g0p.shape), _full(clusT.shape)]
    out_specs = [
        _rows((_BLK, 500)), _rows((_BLK, 500)), _rows((_BLK, 2000)),
        _rows((_BLK, _NZ)), _rows((_BLK, _NC)), _rows((_BLK, _D)),
        pl.BlockSpec((4, _BLK, 128), lambda i: (0, i, 0)),
    ]
    return pl.pallas_call(
        _t1_body, grid=(_GRID,), in_specs=in_specs, out_specs=out_specs,
        out_shape=outs)(*args)


def _make_att(Fh, Cin, Win, Freal, Cout, Wout):
    def body(h_ref, zc_ref, mw, mb, gw, s_o, z_o):
        h = h_ref[...]
        zraw = jnp.concatenate([zc_ref[c] for c in range(Cin)], axis=1)
        zk = _relu(zraw[:, :Freal])
        mwv = mw[...]
        t = _dot(h, mwv[:Fh]) + _dot(zk, mwv[Fh:]) + mb[...]
        pcol = _norm2(_softmax(_leaky(t)))
        comb = pcol[:, 0:1] * zk + pcol[:, 1:2] * h
        s = _dot(comb, gw[...])
        for c in range(Cout):
            s_o[c] = s[:, c * Wout:(c + 1) * Wout]
        z_o[...] = zk

    def run(h, zc, mW, mB, gW):
        outs = [
            jax.ShapeDtypeStruct((Cout, _N, Wout), jnp.float32),
            jax.ShapeDtypeStruct((_N, Freal), jnp.float32),
        ]
        in_specs = [
            _rows((_BLK, Fh)),
            pl.BlockSpec((Cin, _BLK, Win), lambda i: (0, i, 0)),
            _full(mW.shape), _full((1, mB.shape[0])), _full(gW.shape),
        ]
        out_specs = [
            pl.BlockSpec((Cout, _BLK, Wout), lambda i: (0, i, 0)),
            _rows((_BLK, Freal)),
        ]
        return pl.pallas_call(
            body, grid=(_GRID,), in_specs=in_specs, out_specs=out_specs,
            out_shape=outs)(h, zc, mW, mB.reshape(1, -1), gW)
    return run


def _a4_body(z1_ref, z2_ref, z3_ref, z4c_ref, z_ref, mlw, mlb, gzp, s5_o):
    z1 = z1_ref[...]
    z2 = z2_ref[...]
    z3 = z3_ref[...]
    z4 = _relu(z4c_ref[0][:, :_NZ])
    zz = z_ref[...]
    mw = mlw[...]
    t = (_dot(z1, mw[0:500]) + _dot(z2, mw[500:1000]) + _dot(z3, mw[1000:3000])
         + _dot(z4, mw[3000:3010]) + _dot(zz, mw[3010:3020]) + mlb[...])
    w = _norm2(_softmax(_leaky(t)))
    g = gzp[...]
    s5 = (_dot(w[:, 0:1] * z1, g[0:500]) + _dot(w[:, 1:2] * z2, g[500:1000])
          + _dot(w[:, 2:3] * z3, g[1000:3000]) + _dot(w[:, 3:4] * z4, g[3000:3010])
          + _dot(w[:, 4:5] * zz, g[3010:3020]))
    s5_o[0] = s5


def _stage_a4(z1, z2, z3, z4c, z, mLW, mLB, gzp):
    in_specs = [
        _rows((_BLK, 500)), _rows((_BLK, 500)), _rows((_BLK, 2000)),
        pl.BlockSpec((1, _BLK, 16), lambda i: (0, i, 0)),
        _rows((_BLK, _NZ)),
        _full(mLW.shape), _full((1, 5)), _full(gzp.shape),
    ]
    out_specs = [pl.BlockSpec((1, _BLK, 16), lambda i: (0, i, 0))]
    outs = [jax.ShapeDtypeStruct((1, _N, 16), jnp.float32)]
    return pl.pallas_call(
        _a4_body, grid=(_GRID,), in_specs=in_specs, out_specs=out_specs,
        out_shape=outs)(z1, z2, z3, z4c, z, mLW, mLB.reshape(1, -1), gzp)[0]


def _a5_body(noc_ref, q_ref, mqw, mqb, pred_o, net_o, zf_o, loss_o):
    i = pl.program_id(0)
    no = noc_ref[0][:, :_NC]
    pred = _softmax(no)
    q = q_ref[...]
    mw = mqw[...]
    t = _dot(pred, mw[:_NC]) + _dot(q, mw[_NC:]) + mqb[...]
    pzh = _norm2(_softmax(_leaky(t)))
    zf = _softmax(pzh[:, 0:1] * pred + pzh[:, 1:2] * q)
    m = jnp.max(zf, axis=1, keepdims=True)
    ismax = (zf == m).astype(jnp.float32)
    first = jnp.cumsum(ismax, axis=1)
    onehot = jnp.where((ismax > 0.0) & (first == 1.0), 1.0, 0.0)
    wl = (_norm2(zf) >= 0.8).astype(jnp.float32)
    eps = 1e-12
    ln = wl * (onehot * jnp.log(zf + eps)
               + (1.0 - onehot) * jnp.log(1.0 - zf + eps))
    partial = -jnp.sum(ln) / (_N * _NC)

    @pl.when(i == 0)
    def _():
        loss_o[0, 0] = 0.0

    loss_o[0, 0] += partial
    pred_o[...] = pred
    net_o[...] = no
    zf_o[...] = zf


def _stage_a5(noc, q, mQW, mQB):
    in_specs = [
        pl.BlockSpec((1, _BLK, 16), lambda i: (0, i, 0)),
        _rows((_BLK, _NC)),
        _full(mQW.shape), _full((1, 2)),
    ]
    out_specs = [
        _rows((_BLK, _NC)), _rows((_BLK, _NC)), _rows((_BLK, _NC)),
        pl.BlockSpec((1, 1), lambda i: (0, 0)),
    ]
    outs = [
        jax.ShapeDtypeStruct((_N, _NC), jnp.float32),
        jax.ShapeDtypeStruct((_N, _NC), jnp.float32),
        jax.ShapeDtypeStruct((_N, _NC), jnp.float32),
        jax.ShapeDtypeStruct((1, 1), jnp.float32),
    ]
    return pl.pallas_call(
        _a5_body, grid=(_GRID,), in_specs=in_specs, out_specs=out_specs,
        out_shape=outs)(noc, q, mQW, mQB.reshape(1, -1))


def _make_segsum(C, Wc):
    NCI = (C + 1) // 2
    mesh = plsc.VectorSubcoreMesh(core_axis_name="c", subcore_axis_name="s")

    @functools.partial(
        pl.kernel,
        out_type=jax.ShapeDtypeStruct((C * _N, Wc), jnp.float32),
        mesh=mesh,
        scratch_types=[
            pltpu.VMEM((_K,), jnp.int32),
            pltpu.VMEM((_K,), jnp.int32),
            pltpu.VMEM((_K,), jnp.float32),
            pltpu.VMEM((_K, Wc), jnp.float32),
            pltpu.VMEM((_RPT, Wc), jnp.float32),
            pltpu.VMEM_SHARED((_N, Wc), jnp.float32),
            pltpu.SemaphoreType.DMA,
        ],
    )
    def seg(s_hbm, srcadj_hbm, dst_hbm, ew_hbm, out_hbm,
            src_v, dst_v, ew_v, rows_v, zbuf, acc, sem):
        core = lax.axis_index("c")
        tid = lax.axis_index("s")

        def zrow(i, carry):
            for v in range(Wc // 16):
                zbuf[i, pl.ds(v * 16, 16)] = jnp.zeros((16,), jnp.float32)
            return carry
        lax.fori_loop(0, _RPT, zrow, 0)

        for ci in range(NCI):
            c = ci * 2 + core
            valid = c < C

            @pl.when(valid)
            def _():
                pltpu.sync_copy(zbuf, acc.at[pl.ds(tid * _RPT, _RPT)])

            plsc.subcore_barrier()

            @pl.when(valid)
            def _():
                def batch(b, carry):
                    off = pl.multiple_of(tid * _EPT + b * _K, _K)
                    aoff = pl.multiple_of(c * _E_PAD + off, _K)
                    pltpu.sync_copy(srcadj_hbm.at[pl.ds(aoff, _K)], src_v)
                    pltpu.sync_copy(dst_hbm.at[pl.ds(off, _K)], dst_v)
                    pltpu.sync_copy(ew_hbm.at[pl.ds(off, _K)], ew_v)
                    pltpu.async_copy(s_hbm.at[src_v], rows_v, sem).wait()

                    def scale(e, c2):
                        w = plsc.load_gather(
                            ew_v, [jnp.full((16,), e, jnp.int32)])
                        for v in range(Wc // 16):
                            sl = pl.ds(v * 16, 16)
                            rows_v[e, sl] = rows_v[e, sl] * w
                        return c2
                    lax.fori_loop(0, _K, scale, 0)
                    pltpu.sync_copy(rows_v, acc.at[dst_v], add=True)
                    return carry
                lax.fori_loop(0, _NB, batch, 0)

            plsc.subcore_barrier()

            @pl.when(valid)
            def _():
                pltpu.sync_copy(
                    acc.at[pl.ds(tid * _RPT, _RPT)],
                    out_hbm.at[pl.ds(c * _N + tid * _RPT, _RPT)])

            plsc.subcore_barrier()

    return seg


_seg_4_128 = _make_segsum(4, 128)
_seg_16_128 = _make_segsum(16, 128)
_seg_1_16 = _make_segsum(1, 16)

_att1 = _make_att(500, 4, 128, 500, 4, 128)
_att2 = _make_att(500, 4, 128, 500, 16, 128)
_att3 = _make_att(2000, 16, 128, 2000, 1, 16)


def _padcols(w, cols):
    return jnp.pad(w, ((0, 0), (0, cols - w.shape[1])))


def kernel(x, edge_index, edge_weight, params):
    p = params
    src = edge_index[0].astype(jnp.int32)
    dst = edge_index[1].astype(jnp.int32)
    pad = _E_PAD - _E
    src_p = jnp.pad(src, (0, pad))
    dst_p = jnp.pad(dst, (0, pad))
    ew_p = jnp.pad(edge_weight.astype(jnp.float32), (0, pad))

    offs4 = (jnp.arange(4, dtype=jnp.int32) * _N)[:, None]
    srcadj4 = (offs4 + src_p[None, :]).reshape(-1)
    offs16 = (jnp.arange(16, dtype=jnp.int32) * _N)[:, None]
    srcadj16 = (offs16 + src_p[None, :]).reshape(-1)

    g0p = _padcols(p['g0'], 512)
    g1p = _padcols(p['g1'], 512)
    g2p = _padcols(p['g2'], 2048)
    g3p = _padcols(p['g3'], 16)
    gzp = _padcols(p['gz'], 16)
    clusT = p['cluster'].T

    h1, h2, h3, z, q, x_bar, s1c = _stage_t1(x, p, clusT, g0p)

    seg1 = _seg_4_128(s1c.reshape(4 * _N, 128), srcadj4, dst_p, ew_p)
    s2c, z1 = _att1(h1, seg1.reshape(4, _N, 128), p['m1'][0], p['m1'][1], g1p)

    seg2 = _seg_4_128(s2c.reshape(4 * _N, 128), srcadj4, dst_p, ew_p)
    s3c, z2 = _att2(h2, seg2.reshape(4, _N, 128), p['m2'][0], p['m2'][1], g2p)

    seg3 = _seg_16_128(s3c.reshape(16 * _N, 128), srcadj16, dst_p, ew_p)
    s4c, z3 = _att3(h3, seg3.reshape(16, _N, 128), p['m3'][0], p['m3'][1], g3p)

    seg4 = _seg_1_16(s4c.reshape(_N, 16), src_p, dst_p, ew_p)
    s5c = _stage_a4(z1, z2, z3, seg4.reshape(1, _N, 16), z,
                    p['mL'][0], p['mL'][1], gzp)

    seg5 = _seg_1_16(s5c.reshape(_N, 16), src_p, dst_p, ew_p)
    predict, net_output, z_F, loss = _stage_a5(
        seg5.reshape(1, _N, 16), q, p['mZQ'][0], p['mZQ'][1])

    return (x_bar, q, predict, z, net_output, loss[0, 0], z_F)


# speedup vs baseline: 1.4043x; 1.4043x over previous
import functools

import jax
import jax.numpy as jnp
from jax import lax
from jax.experimental import pallas as pl
from jax.experimental.pallas import tpu as pltpu
from jax.experimental.pallas import tpu_sc as plsc

_N = 10000
_E = 160000
_D = 128
_NZ = 10
_NC = 10
_V = 1.0

_BLK = 1000
_GRID = _N // _BLK

_TILES = 16
_K = 128
_E_PAD = ((_E + _TILES * _K - 1) // (_TILES * _K)) * (_TILES * _K)
_EPT = _E_PAD // _TILES
_NB = _EPT // _K
_RA = 624
_RB = _N - 15 * _RA

_HI = lax.Precision.HIGHEST


def _dot(a, b):
    return jnp.dot(a, b, precision=_HI, preferred_element_type=jnp.float32)


def _relu(a):
    return jnp.maximum(a, 0.0)


def _leaky(a):
    return jnp.where(a > 0, a, 0.01 * a)


def _softmax(a):
    m = jnp.max(a, axis=1, keepdims=True)
    e = jnp.exp(a - m)
    return e / jnp.sum(e, axis=1, keepdims=True)


def _norm2(a):
    n = jnp.sqrt(jnp.sum(a * a, axis=1, keepdims=True))
    return a / jnp.maximum(n, 1e-12)


def _full(shape):
    return pl.BlockSpec(shape, lambda i: (0,) * len(shape))


def _rows(shape):
    return pl.BlockSpec(shape, lambda i: (i,) + (0,) * (len(shape) - 1))


def _t1_body(x_ref, w1, b1, w2, b2, w3, b3, wz, bz, wd1, bd1, wd2, bd2,
             wd3, bd3, wx, bx, g0p, clusT,
             h1_o, h2_o, h3_o, z_o, q_o, xbar_o, s1_o):
    x = x_ref[...]
    h1 = _relu(_dot(x, w1[...]) + b1[...])
    h2 = _relu(_dot(h1, w2[...]) + b2[...])
    h3 = _relu(_dot(h2, w3[...]) + b3[...])
    z = _dot(h3, wz[...]) + bz[...]
    d1 = _relu(_dot(z, wd1[...]) + bd1[...])
    d2 = _relu(_dot(d1, wd2[...]) + bd2[...])
    d3 = _relu(_dot(d2, wd3[...]) + bd3[...])
    xbar_o[...] = _dot(d3, wx[...]) + bx[...]

    ct = clusT[...]
    z2s = jnp.sum(z * z, axis=1, keepdims=True)
    c2s = jnp.sum(ct * ct, axis=0)[None, :]
    dist = z2s + c2s - 2.0 * _dot(z, ct)
    q = 1.0 / (1.0 + dist / _V)
    q_o[...] = q / jnp.sum(q, axis=1, keepdims=True)

    s1 = _dot(x, g0p[...])
    for c in range(4):
        s1_o[c] = s1[:, c * 128:(c + 1) * 128]

    h1_o[...] = h1
    h2_o[...] = h2
    h3_o[...] = h3
    z_o[...] = z


def _stage_t1(x, p, clusT, g0p):
    B = 400
    G = _N // B
    outs = [
        jax.ShapeDtypeStruct((_N, 500), jnp.float32),
        jax.ShapeDtypeStruct((_N, 500), jnp.float32),
        jax.ShapeDtypeStruct((_N, 2000), jnp.float32),
        jax.ShapeDtypeStruct((_N, _NZ), jnp.float32),
        jax.ShapeDtypeStruct((_N, _NC), jnp.float32),
        jax.ShapeDtypeStruct((_N, _D), jnp.float32),
        jax.ShapeDtypeStruct((4, _N, 128), jnp.float32),
    ]
    args = [x]
    in_specs = [_rows((B, _D))]
    for name in ('enc1', 'enc2', 'enc3', 'zl', 'dec1', 'dec2', 'dec3', 'xbar'):
        w, b = p[name]
        args += [w, b.reshape(1, -1)]
        in_specs += [_full(w.shape), _full((1, b.shape[0]))]
    args += [g0p, clusT]
    in_specs += [_full(g0p.shape), _full(clusT.shape)]
    out_specs = [
        _rows((B, 500)), _rows((B, 500)), _rows((B, 2000)),
        _rows((B, _NZ)), _rows((B, _NC)), _rows((B, _D)),
        pl.BlockSpec((4, B, 128), lambda i: (0, i, 0)),
    ]
    return pl.pallas_call(
        _t1_body, grid=(G,), in_specs=in_specs, out_specs=out_specs,
        out_shape=outs)(*args)


def _make_att(Fh, Cin, Win, Freal, Cout, Wout, B=_BLK):
    def body(h_ref, zc_ref, mw, mb, gw, s_o, z_o):
        h = h_ref[...]
        zraw = jnp.concatenate([zc_ref[c] for c in range(Cin)], axis=1)
        zk = _relu(zraw[:, :Freal])
        mwv = mw[...]
        t = _dot(h, mwv[:Fh]) + _dot(zk, mwv[Fh:]) + mb[...]
        pcol = _norm2(_softmax(_leaky(t)))
        comb = pcol[:, 0:1] * zk + pcol[:, 1:2] * h
        s = _dot(comb, gw[...])
        for c in range(Cout):
            s_o[c] = s[:, c * Wout:(c + 1) * Wout]
        z_o[...] = zk

    def run(h, zc, mW, mB, gW):
        outs = [
            jax.ShapeDtypeStruct((Cout, _N, Wout), jnp.float32),
            jax.ShapeDtypeStruct((_N, Freal), jnp.float32),
        ]
        in_specs = [
            _rows((B, Fh)),
            pl.BlockSpec((Cin, B, Win), lambda i: (0, i, 0)),
            _full(mW.shape), _full((1, mB.shape[0])), _full(gW.shape),
        ]
        out_specs = [
            pl.BlockSpec((Cout, B, Wout), lambda i: (0, i, 0)),
            _rows((B, Freal)),
        ]
        return pl.pallas_call(
            body, grid=(_N // B,), in_specs=in_specs, out_specs=out_specs,
            out_shape=outs)(h, zc, mW, mB.reshape(1, -1), gW)
    return run


def _a4_body(z1_ref, z2_ref, z3_ref, z4c_ref, z_ref, mlw, mlb, gzp, s5_o):
    z1 = z1_ref[...]
    z2 = z2_ref[...]
    z3 = z3_ref[...]
    z4 = _relu(z4c_ref[0][:, :_NZ])
    zz = z_ref[...]
    mw = mlw[...]
    t = (_dot(z1, mw[0:500]) + _dot(z2, mw[500:1000]) + _dot(z3, mw[1000:3000])
         + _dot(z4, mw[3000:3010]) + _dot(zz, mw[3010:3020]) + mlb[...])
    w = _norm2(_softmax(_leaky(t)))
    g = gzp[...]
    s5 = (_dot(w[:, 0:1] * z1, g[0:500]) + _dot(w[:, 1:2] * z2, g[500:1000])
          + _dot(w[:, 2:3] * z3, g[1000:3000]) + _dot(w[:, 3:4] * z4, g[3000:3010])
          + _dot(w[:, 4:5] * zz, g[3010:3020]))
    s5_o[0] = s5


def _stage_a4(z1, z2, z3, z4c, z, mLW, mLB, gzp):
    in_specs = [
        _rows((_BLK, 500)), _rows((_BLK, 500)), _rows((_BLK, 2000)),
        pl.BlockSpec((1, _BLK, 128), lambda i: (0, i, 0)),
        _rows((_BLK, _NZ)),
        _full(mLW.shape), _full((1, 5)), _full(gzp.shape),
    ]
    out_specs = [pl.BlockSpec((1, _BLK, 128), lambda i: (0, i, 0))]
    outs = [jax.ShapeDtypeStruct((1, _N, 128), jnp.float32)]
    return pl.pallas_call(
        _a4_body, grid=(_GRID,), in_specs=in_specs, out_specs=out_specs,
        out_shape=outs)(z1, z2, z3, z4c, z, mLW, mLB.reshape(1, -1), gzp)[0]


def _a5_body(noc_ref, q_ref, mqw, mqb, pred_o, net_o, zf_o, loss_o):
    i = pl.program_id(0)
    no = noc_ref[0][:, :_NC]
    pred = _softmax(no)
    q = q_ref[...]
    mw = mqw[...]
    t = _dot(pred, mw[:_NC]) + _dot(q, mw[_NC:]) + mqb[...]
    pzh = _norm2(_softmax(_leaky(t)))
    zf = _softmax(pzh[:, 0:1] * pred + pzh[:, 1:2] * q)
    m = jnp.max(zf, axis=1, keepdims=True)
    col = lax.broadcasted_iota(jnp.int32, zf.shape, 1).astype(jnp.float32)
    amax = jnp.min(jnp.where(zf == m, col, 1e9), axis=1, keepdims=True)
    onehot = jnp.where(col == amax, 1.0, 0.0)
    wl = (_norm2(zf) >= 0.8).astype(jnp.float32)
    eps = 1e-12
    ln = wl * (onehot * jnp.log(zf + eps)
               + (1.0 - onehot) * jnp.log(1.0 - zf + eps))
    partial = -jnp.sum(ln) / (_N * _NC)

    @pl.when(i == 0)
    def _():
        loss_o[...] = jnp.zeros((1, 1), jnp.float32)

    loss_o[...] += jnp.full((1, 1), partial, jnp.float32)
    pred_o[...] = pred
    net_o[...] = no
    zf_o[...] = zf


def _stage_a5(noc, q, mQW, mQB):
    in_specs = [
        pl.BlockSpec((1, _BLK, 128), lambda i: (0, i, 0)),
        _rows((_BLK, _NC)),
        _full(mQW.shape), _full((1, 2)),
    ]
    out_specs = [
        _rows((_BLK, _NC)), _rows((_BLK, _NC)), _rows((_BLK, _NC)),
        pl.BlockSpec((1, 1), lambda i: (0, 0)),
    ]
    outs = [
        jax.ShapeDtypeStruct((_N, _NC), jnp.float32),
        jax.ShapeDtypeStruct((_N, _NC), jnp.float32),
        jax.ShapeDtypeStruct((_N, _NC), jnp.float32),
        jax.ShapeDtypeStruct((1, 1), jnp.float32),
    ]
    return pl.pallas_call(
        _a5_body, grid=(_GRID,), in_specs=in_specs, out_specs=out_specs,
        out_shape=outs)(noc, q, mQW, mQB.reshape(1, -1))


@functools.lru_cache(maxsize=None)
def _make_segsum(C, Wc):
    NCI = (C + 1) // 2
    mesh = plsc.VectorSubcoreMesh(core_axis_name="c", subcore_axis_name="s",
                                  num_cores=2, num_subcores=_TILES)

    @functools.partial(
        pl.kernel,
        out_type=jax.ShapeDtypeStruct((C * _N, Wc), jnp.float32),
        mesh=mesh,
        scratch_types=[
            pltpu.VMEM((_K,), jnp.int32),
            pltpu.VMEM((_K,), jnp.int32),
            pltpu.VMEM((_K, 16), jnp.float32),
            pltpu.VMEM((_K, Wc), jnp.float32),
            pltpu.VMEM((16, Wc), jnp.float32),
            pltpu.VMEM_SHARED((_N, Wc), jnp.float32),
            pltpu.SemaphoreType.DMA,
        ],
    )
    def seg(s_hbm, srcadj_hbm, dst_hbm, ew_hbm, out_hbm,
            src_v, dst_v, ew_v, rows_v, zbuf, acc, sem):
        core = lax.axis_index("c")
        tid = lax.axis_index("s")

        def zrow(i, carry):
            for v in range(Wc // 16):
                zbuf[i, pl.ds(v * 16, 16)] = jnp.zeros((16,), jnp.float32)
            return carry
        lax.fori_loop(0, 16, zrow, 0)
        last = tid == _TILES - 1
        zstart = tid * _RA
        znum = jnp.where(last, _RB // 16, _RA // 16)

        for ci in range(NCI):
            c = ci * 2 + core
            valid = c < C

            @pl.when(valid)
            def _():
                def zchunk(i, carry):
                    pltpu.sync_copy(zbuf, acc.at[pl.ds(zstart + i * 16, 16)])
                    return carry
                lax.fori_loop(0, znum, zchunk, 0)

            plsc.subcore_barrier()

            @pl.when(valid)
            def _():
                def batch(b, carry):
                    off = pl.multiple_of(tid * _EPT + b * _K, _K)
                    aoff = pl.multiple_of(c * _E_PAD + off, _K)
                    pltpu.sync_copy(srcadj_hbm.at[pl.ds(aoff, _K)], src_v)
                    pltpu.sync_copy(dst_hbm.at[pl.ds(off, _K)], dst_v)
                    pltpu.sync_copy(ew_hbm.at[pl.ds(off, _K)], ew_v)
                    pltpu.async_copy(s_hbm.at[src_v], rows_v, sem).wait()

                    def scale(e, c2):
                        w = ew_v[e, :]
                        for v in range(Wc // 16):
                            sl = pl.ds(v * 16, 16)
                            rows_v[e, sl] = rows_v[e, sl] * w
                        return c2
                    lax.fori_loop(0, _K, scale, 0)
                    pltpu.sync_copy(rows_v, acc.at[dst_v], add=True)
                    return carry
                lax.fori_loop(0, _NB, batch, 0)

            plsc.subcore_barrier()

            @pl.when(valid & jnp.logical_not(last))
            def _():
                pltpu.sync_copy(
                    acc.at[pl.ds(tid * _RA, _RA)],
                    out_hbm.at[pl.ds(c * _N + tid * _RA, _RA)])

            @pl.when(valid & last)
            def _():
                pltpu.sync_copy(
                    acc.at[pl.ds(15 * _RA, _RB)],
                    out_hbm.at[pl.ds(c * _N + 15 * _RA, _RB)])

            plsc.subcore_barrier()

    return seg


_att1 = _make_att(500, 4, 128, 500, 4, 128)
_att2 = _make_att(500, 4, 128, 500, 16, 128)
_att3 = _make_att(2000, 16, 128, 2000, 1, 128, B=400)


def _padcols(w, cols):
    return jnp.pad(w, ((0, 0), (0, cols - w.shape[1])))


def kernel(x, edge_index, edge_weight, params):
    p = params
    src = edge_index[0].astype(jnp.int32)
    dst = edge_index[1].astype(jnp.int32)
    pad = _E_PAD - _E
    src_p = jnp.pad(src, (0, pad))
    dst_p = jnp.pad(dst, (0, pad))
    ew_p = jnp.pad(edge_weight.astype(jnp.float32), (0, pad))
    ewx = jnp.broadcast_to(ew_p[:, None], (_E_PAD, 16))

    offs4 = (jnp.arange(4, dtype=jnp.int32) * _N)[:, None]
    srcadj4 = (offs4 + src_p[None, :]).reshape(-1)
    offs16 = (jnp.arange(16, dtype=jnp.int32) * _N)[:, None]
    srcadj16 = (offs16 + src_p[None, :]).reshape(-1)

    g0p = _padcols(p['g0'], 512)
    g1p = _padcols(p['g1'], 512)
    g2p = _padcols(p['g2'], 2048)
    g3p = _padcols(p['g3'], 128)
    gzp = _padcols(p['gz'], 128)
    clusT = p['cluster'].T

    h1, h2, h3, z, q, x_bar, s1c = _stage_t1(x, p, clusT, g0p)

    seg1 = _make_segsum(4, 128)(s1c.reshape(4 * _N, 128), srcadj4, dst_p, ewx)
    s2c, z1 = _att1(h1, seg1.reshape(4, _N, 128), p['m1'][0], p['m1'][1], g1p)

    seg2 = _make_segsum(4, 128)(s2c.reshape(4 * _N, 128), srcadj4, dst_p, ewx)
    s3c, z2 = _att2(h2, seg2.reshape(4, _N, 128), p['m2'][0], p['m2'][1], g2p)

    seg3 = _make_segsum(16, 128)(s3c.reshape(16 * _N, 128), srcadj16, dst_p, ewx)
    s4c, z3 = _att3(h3, seg3.reshape(16, _N, 128), p['m3'][0], p['m3'][1], g3p)

    seg4 = _make_segsum(1, 128)(s4c.reshape(_N, 128), src_p, dst_p, ewx)
    s5c = _stage_a4(z1, z2, z3, seg4.reshape(1, _N, 128), z,
                    p['mL'][0], p['mL'][1], gzp)

    seg5 = _make_segsum(1, 128)(s5c.reshape(_N, 128), src_p, dst_p, ewx)
    predict, net_output, z_F, loss = _stage_a5(
        seg5.reshape(1, _N, 128), q, p['mZQ'][0], p['mZQ'][1])

    return (x_bar, q, predict, z, net_output, loss[0, 0], z_F)


# speedup vs baseline: 2.1126x; 1.5043x over previous
import functools

import jax
import jax.numpy as jnp
from jax import lax
from jax.experimental import pallas as pl
from jax.experimental.pallas import tpu as pltpu
from jax.experimental.pallas import tpu_sc as plsc

_N = 10000
_E = 160000
_D = 128
_NZ = 10
_NC = 10
_V = 1.0

_BLK = 1000
_GRID = _N // _BLK

_TILES = 16
_K = 80
_NB = 126
_EPT = _NB * _K
_E_PAD = _EPT * _TILES
_RA = 624
_RB = _N - 15 * _RA

_HI = lax.Precision.HIGHEST


def _dot(a, b):
    return jnp.dot(a, b, precision=_HI, preferred_element_type=jnp.float32)


def _relu(a):
    return jnp.maximum(a, 0.0)


def _leaky(a):
    return jnp.where(a > 0, a, 0.01 * a)


def _softmax(a):
    m = jnp.max(a, axis=1, keepdims=True)
    e = jnp.exp(a - m)
    return e / jnp.sum(e, axis=1, keepdims=True)


def _norm2(a):
    n = jnp.sqrt(jnp.sum(a * a, axis=1, keepdims=True))
    return a / jnp.maximum(n, 1e-12)


def _full(shape):
    return pl.BlockSpec(shape, lambda i: (0,) * len(shape))


def _rows(shape):
    return pl.BlockSpec(shape, lambda i: (i,) + (0,) * (len(shape) - 1))


def _t1_body(x_ref, w1, b1, w2, b2, w3, b3, wz, bz, wd1, bd1, wd2, bd2,
             wd3, bd3, wx, bx, g0p, clusT,
             h1_o, h2_o, h3_o, z_o, q_o, xbar_o, s1_o):
    x = x_ref[...]
    h1 = _relu(_dot(x, w1[...]) + b1[...])
    h2 = _relu(_dot(h1, w2[...]) + b2[...])
    h3 = _relu(_dot(h2, w3[...]) + b3[...])
    z = _dot(h3, wz[...]) + bz[...]
    d1 = _relu(_dot(z, wd1[...]) + bd1[...])
    d2 = _relu(_dot(d1, wd2[...]) + bd2[...])
    d3 = _relu(_dot(d2, wd3[...]) + bd3[...])
    xbar_o[...] = _dot(d3, wx[...]) + bx[...]

    ct = clusT[...]
    z2s = jnp.sum(z * z, axis=1, keepdims=True)
    c2s = jnp.sum(ct * ct, axis=0)[None, :]
    dist = z2s + c2s - 2.0 * _dot(z, ct)
    q = 1.0 / (1.0 + dist / _V)
    q_o[...] = q / jnp.sum(q, axis=1, keepdims=True)

    s1 = _dot(x, g0p[...])
    for c in range(4):
        s1_o[c] = s1[:, c * 128:(c + 1) * 128]

    h1_o[...] = h1
    h2_o[...] = h2
    h3_o[...] = h3
    z_o[...] = z


def _stage_t1(x, p, clusT, g0p):
    B = 400
    G = _N // B
    outs = [
        jax.ShapeDtypeStruct((_N, 500), jnp.float32),
        jax.ShapeDtypeStruct((_N, 500), jnp.float32),
        jax.ShapeDtypeStruct((_N, 2000), jnp.float32),
        jax.ShapeDtypeStruct((_N, _NZ), jnp.float32),
        jax.ShapeDtypeStruct((_N, _NC), jnp.float32),
        jax.ShapeDtypeStruct((_N, _D), jnp.float32),
        jax.ShapeDtypeStruct((4, _N, 128), jnp.float32),
    ]
    args = [x]
    in_specs = [_rows((B, _D))]
    for name in ('enc1', 'enc2', 'enc3', 'zl', 'dec1', 'dec2', 'dec3', 'xbar'):
        w, b = p[name]
        args += [w, b.reshape(1, -1)]
        in_specs += [_full(w.shape), _full((1, b.shape[0]))]
    args += [g0p, clusT]
    in_specs += [_full(g0p.shape), _full(clusT.shape)]
    out_specs = [
        _rows((B, 500)), _rows((B, 500)), _rows((B, 2000)),
        _rows((B, _NZ)), _rows((B, _NC)), _rows((B, _D)),
        pl.BlockSpec((4, B, 128), lambda i: (0, i, 0)),
    ]
    return pl.pallas_call(
        _t1_body, grid=(G,), in_specs=in_specs, out_specs=out_specs,
        out_shape=outs)(*args)


def _make_att(Fh, Cin, Win, Freal, Cout, Wout, B=_BLK):
    def body(h_ref, zc_ref, mw, mb, gw, s_o, z_o):
        h = h_ref[...]
        zraw = jnp.concatenate([zc_ref[c] for c in range(Cin)], axis=1)
        zk = _relu(zraw[:, :Freal])
        mwv = mw[...]
        t = _dot(h, mwv[:Fh]) + _dot(zk, mwv[Fh:]) + mb[...]
        pcol = _norm2(_softmax(_leaky(t)))
        comb = pcol[:, 0:1] * zk + pcol[:, 1:2] * h
        s = _dot(comb, gw[...])
        for c in range(Cout):
            s_o[c] = s[:, c * Wout:(c + 1) * Wout]
        z_o[...] = zk

    def run(h, zc, mW, mB, gW):
        outs = [
            jax.ShapeDtypeStruct((Cout, _N, Wout), jnp.float32),
            jax.ShapeDtypeStruct((_N, Freal), jnp.float32),
        ]
        in_specs = [
            _rows((B, Fh)),
            pl.BlockSpec((Cin, B, Win), lambda i: (0, i, 0)),
            _full(mW.shape), _full((1, mB.shape[0])), _full(gW.shape),
        ]
        out_specs = [
            pl.BlockSpec((Cout, B, Wout), lambda i: (0, i, 0)),
            _rows((B, Freal)),
        ]
        return pl.pallas_call(
            body, grid=(_N // B,), in_specs=in_specs, out_specs=out_specs,
            out_shape=outs)(h, zc, mW, mB.reshape(1, -1), gW)
    return run


def _a4_body(z1_ref, z2_ref, z3_ref, z4c_ref, z_ref, mlw, mlb, gzp, s5_o):
    z1 = z1_ref[...]
    z2 = z2_ref[...]
    z3 = z3_ref[...]
    z4 = _relu(z4c_ref[0][:, :_NZ])
    zz = z_ref[...]
    mw = mlw[...]
    t = (_dot(z1, mw[0:500]) + _dot(z2, mw[500:1000]) + _dot(z3, mw[1000:3000])
         + _dot(z4, mw[3000:3010]) + _dot(zz, mw[3010:3020]) + mlb[...])
    w = _norm2(_softmax(_leaky(t)))
    g = gzp[...]
    s5 = (_dot(w[:, 0:1] * z1, g[0:500]) + _dot(w[:, 1:2] * z2, g[500:1000])
          + _dot(w[:, 2:3] * z3, g[1000:3000]) + _dot(w[:, 3:4] * z4, g[3000:3010])
          + _dot(w[:, 4:5] * zz, g[3010:3020]))
    s5_o[0] = s5


def _stage_a4(z1, z2, z3, z4c, z, mLW, mLB, gzp):
    in_specs = [
        _rows((_BLK, 500)), _rows((_BLK, 500)), _rows((_BLK, 2000)),
        pl.BlockSpec((1, _BLK, 128), lambda i: (0, i, 0)),
        _rows((_BLK, _NZ)),
        _full(mLW.shape), _full((1, 5)), _full(gzp.shape),
    ]
    out_specs = [pl.BlockSpec((1, _BLK, 128), lambda i: (0, i, 0))]
    outs = [jax.ShapeDtypeStruct((1, _N, 128), jnp.float32)]
    return pl.pallas_call(
        _a4_body, grid=(_GRID,), in_specs=in_specs, out_specs=out_specs,
        out_shape=outs)(z1, z2, z3, z4c, z, mLW, mLB.reshape(1, -1), gzp)[0]


def _a5_body(noc_ref, q_ref, mqw, mqb, pred_o, net_o, zf_o, loss_o):
    i = pl.program_id(0)
    no = noc_ref[0][:, :_NC]
    pred = _softmax(no)
    q = q_ref[...]
    mw = mqw[...]
    t = _dot(pred, mw[:_NC]) + _dot(q, mw[_NC:]) + mqb[...]
    pzh = _norm2(_softmax(_leaky(t)))
    zf = _softmax(pzh[:, 0:1] * pred + pzh[:, 1:2] * q)
    m = jnp.max(zf, axis=1, keepdims=True)
    col = lax.broadcasted_iota(jnp.int32, zf.shape, 1).astype(jnp.float32)
    amax = jnp.min(jnp.where(zf == m, col, 1e9), axis=1, keepdims=True)
    onehot = jnp.where(col == amax, 1.0, 0.0)
    wl = (_norm2(zf) >= 0.8).astype(jnp.float32)
    eps = 1e-12
    ln = wl * (onehot * jnp.log(zf + eps)
               + (1.0 - onehot) * jnp.log(1.0 - zf + eps))
    partial = -jnp.sum(ln) / (_N * _NC)

    @pl.when(i == 0)
    def _():
        loss_o[...] = jnp.zeros((1, 1), jnp.float32)

    loss_o[...] += jnp.full((1, 1), partial, jnp.float32)
    pred_o[...] = pred
    net_o[...] = no
    zf_o[...] = zf


def _stage_a5(noc, q, mQW, mQB):
    in_specs = [
        pl.BlockSpec((1, _BLK, 128), lambda i: (0, i, 0)),
        _rows((_BLK, _NC)),
        _full(mQW.shape), _full((1, 2)),
    ]
    out_specs = [
        _rows((_BLK, _NC)), _rows((_BLK, _NC)), _rows((_BLK, _NC)),
        pl.BlockSpec((1, 1), lambda i: (0, 0)),
    ]
    outs = [
        jax.ShapeDtypeStruct((_N, _NC), jnp.float32),
        jax.ShapeDtypeStruct((_N, _NC), jnp.float32),
        jax.ShapeDtypeStruct((_N, _NC), jnp.float32),
        jax.ShapeDtypeStruct((1, 1), jnp.float32),
    ]
    return pl.pallas_call(
        _a5_body, grid=(_GRID,), in_specs=in_specs, out_specs=out_specs,
        out_shape=outs)(noc, q, mQW, mQB.reshape(1, -1))


@functools.lru_cache(maxsize=None)
def _make_segsum(C, Wc):
    NCI = (C + 1) // 2
    mesh = plsc.VectorSubcoreMesh(core_axis_name="c", subcore_axis_name="s",
                                  num_cores=2, num_subcores=_TILES)

    @functools.partial(
        pl.kernel,
        out_type=jax.ShapeDtypeStruct((C * _N, Wc), jnp.float32),
        mesh=mesh,
        scratch_types=[
            [pltpu.VMEM((_K,), jnp.int32)] * 2,
            [pltpu.VMEM((_K,), jnp.int32)] * 2,
            [pltpu.VMEM((_K,), jnp.int32)] * 2,
            [pltpu.VMEM((_K, 16), jnp.float32)] * 2,
            [pltpu.VMEM((_K, Wc), jnp.float32)] * 2,
            pltpu.VMEM((16, Wc), jnp.float32),
            pltpu.VMEM_SHARED((_N, Wc), jnp.float32),
            [pltpu.SemaphoreType.DMA] * 2,
            [pltpu.SemaphoreType.DMA] * 2,
            [pltpu.SemaphoreType.DMA] * 2,
            pltpu.SemaphoreType.DMA,
        ],
    )
    def seg(s_hbm, srcadj_hbm, dst_hbm, ew_hbm, out_hbm,
            srca, dstv, dscat, ewv, rows, zbuf, acc,
            sem_i, sem_g, sem_s, sem_z):
        core = lax.axis_index("c")
        tid = lax.axis_index("s")

        def zrow(i, carry):
            for v in range(Wc // 16):
                zbuf[i, pl.ds(v * 16, 16)] = jnp.zeros((16,), jnp.float32)
            return carry
        lax.fori_loop(0, 16, zrow, 0)
        last = tid == _TILES - 1
        zstart = tid * _RA

        def load_idx(c, bi, slot):
            off = pl.multiple_of(tid * _EPT + bi * _K, 8)
            aoff = pl.multiple_of(c * _E_PAD + off, 8)
            pltpu.async_copy(srcadj_hbm.at[pl.ds(aoff, _K)], srca[slot],
                             sem_i[slot])
            pltpu.async_copy(dst_hbm.at[pl.ds(off, _K)], dstv[slot],
                             sem_i[slot])
            pltpu.async_copy(ew_hbm.at[pl.ds(off, _K)], ewv[slot],
                             sem_i[slot])

        def wait_idx(slot):
            pltpu.make_async_copy(srcadj_hbm.at[pl.ds(0, _K)], srca[slot],
                                  sem_i[slot]).wait()
            pltpu.make_async_copy(dst_hbm.at[pl.ds(0, _K)], dstv[slot],
                                  sem_i[slot]).wait()
            pltpu.make_async_copy(ew_hbm.at[pl.ds(0, _K)], ewv[slot],
                                  sem_i[slot]).wait()

        def issue_gather(slot):
            pltpu.async_copy(s_hbm.at[srca[slot]], rows[slot], sem_g[slot])

        def wait_gather(slot):
            pltpu.make_async_copy(s_hbm.at[srca[slot]], rows[slot],
                                  sem_g[slot]).wait()

        def issue_scatter(slot):
            pltpu.async_copy(rows[slot], acc.at[dscat[slot]], sem_s[slot],
                             add=True)

        def wait_scatter(slot):
            pltpu.make_async_copy(rows[slot], acc.at[dscat[slot]],
                                  sem_s[slot]).wait()

        for ci in range(NCI):
            c = ci * 2 + core
            valid = c < C

            @pl.when(valid)
            def _():
                zds = []
                for i in range(_RB // 16):
                    zds.append(pltpu.async_copy(
                        zbuf, acc.at[pl.ds(zstart + i * 16, 16)], sem_z))
                for d in zds:
                    d.wait()

            plsc.subcore_barrier()

            @pl.when(valid)
            def _():
                load_idx(c, 0, 0)
                wait_idx(0)
                issue_gather(0)
                load_idx(c, 1, 1)

                def body2(i, carry):
                    for ph in (0, 1):
                        b = i * 2 + ph
                        p, q = ph, 1 - ph
                        wait_idx(q)

                        @pl.when(b >= 1)
                        def _():
                            wait_scatter(q)

                        issue_gather(q)
                        wait_gather(p)

                        def scale(e, c2):
                            w = ewv[p][e, :]
                            for v in range(Wc // 16):
                                sl = pl.ds(v * 16, 16)
                                rows[p][e, sl] = rows[p][e, sl] * w
                            return c2
                        lax.fori_loop(0, _K, scale, 0)
                        for v in range(_K // 16):
                            sl = pl.ds(v * 16, 16)
                            dscat[p][sl] = dstv[p][sl]
                        load_idx(c, jnp.minimum(b + 2, _NB - 1), p)
                        issue_scatter(p)
                    return carry
                lax.fori_loop(0, _NB // 2, body2, 0)
                wait_gather(0)
                wait_idx(1)
                wait_scatter(1)

            plsc.subcore_barrier()

            @pl.when(valid & jnp.logical_not(last))
            def _():
                pltpu.sync_copy(
                    acc.at[pl.ds(tid * _RA, _RA)],
                    out_hbm.at[pl.ds(c * _N + tid * _RA, _RA)])

            @pl.when(valid & last)
            def _():
                pltpu.sync_copy(
                    acc.at[pl.ds(15 * _RA, _RB)],
                    out_hbm.at[pl.ds(c * _N + 15 * _RA, _RB)])

            plsc.subcore_barrier()

    return seg


_att1 = _make_att(500, 4, 128, 500, 4, 128)
_att2 = _make_att(500, 4, 128, 500, 16, 128)
_att3 = _make_att(2000, 16, 128, 2000, 1, 128, B=400)


def _padcols(w, cols):
    return jnp.pad(w, ((0, 0), (0, cols - w.shape[1])))


def kernel(x, edge_index, edge_weight, params):
    p = params
    src = edge_index[0].astype(jnp.int32)
    dst = edge_index[1].astype(jnp.int32)
    pad = _E_PAD - _E
    src_p = jnp.pad(src, (0, pad))
    dst_p = jnp.pad(dst, (0, pad))
    ew_p = jnp.pad(edge_weight.astype(jnp.float32), (0, pad))
    ewx = jnp.broadcast_to(ew_p[:, None], (_E_PAD, 16))

    offs4 = (jnp.arange(4, dtype=jnp.int32) * _N)[:, None]
    srcadj4 = (offs4 + src_p[None, :]).reshape(-1)
    offs16 = (jnp.arange(16, dtype=jnp.int32) * _N)[:, None]
    srcadj16 = (offs16 + src_p[None, :]).reshape(-1)

    g0p = _padcols(p['g0'], 512)
    g1p = _padcols(p['g1'], 512)
    g2p = _padcols(p['g2'], 2048)
    g3p = _padcols(p['g3'], 128)
    gzp = _padcols(p['gz'], 128)
    clusT = p['cluster'].T

    h1, h2, h3, z, q, x_bar, s1c = _stage_t1(x, p, clusT, g0p)

    seg1 = _make_segsum(4, 128)(s1c.reshape(4 * _N, 128), srcadj4, dst_p, ewx)
    s2c, z1 = _att1(h1, seg1.reshape(4, _N, 128), p['m1'][0], p['m1'][1], g1p)

    seg2 = _make_segsum(4, 128)(s2c.reshape(4 * _N, 128), srcadj4, dst_p, ewx)
    s3c, z2 = _att2(h2, seg2.reshape(4, _N, 128), p['m2'][0], p['m2'][1], g2p)

    seg3 = _make_segsum(16, 128)(s3c.reshape(16 * _N, 128), srcadj16, dst_p, ewx)
    s4c, z3 = _att3(h3, seg3.reshape(16, _N, 128), p['m3'][0], p['m3'][1], g3p)

    seg4 = _make_segsum(1, 128)(s4c.reshape(_N, 128), src_p, dst_p, ewx)
    s5c = _stage_a4(z1, z2, z3, seg4.reshape(1, _N, 128), z,
                    p['mL'][0], p['mL'][1], gzp)

    seg5 = _make_segsum(1, 128)(s5c.reshape(_N, 128), src_p, dst_p, ewx)
    predict, net_output, z_F, loss = _stage_a5(
        seg5.reshape(1, _N, 128), q, p['mZQ'][0], p['mZQ'][1])

    return (x_bar, q, predict, z, net_output, loss[0, 0], z_F)


# speedup vs baseline: 3.2447x; 1.5359x over previous
import functools

import jax
import jax.numpy as jnp
from jax import lax
from jax.experimental import pallas as pl
from jax.experimental.pallas import tpu as pltpu
from jax.experimental.pallas import tpu_sc as plsc

_N = 10000
_E = 160000
_D = 128
_NZ = 10
_NC = 10
_V = 1.0

_BLK = 1000
_GRID = _N // _BLK

_TILES = 16
_K = 80
_NB = 126
_EPT = _NB * _K
_E_PAD = _EPT * _TILES
_RA = 624
_RB = _N - 15 * _RA

_HI = lax.Precision.DEFAULT


def _dot(a, b):
    return jnp.dot(a, b, precision=_HI, preferred_element_type=jnp.float32)


def _relu(a):
    return jnp.maximum(a, 0.0)


def _leaky(a):
    return jnp.where(a > 0, a, 0.01 * a)


def _softmax(a):
    m = jnp.max(a, axis=1, keepdims=True)
    e = jnp.exp(a - m)
    return e / jnp.sum(e, axis=1, keepdims=True)


def _norm2(a):
    n = jnp.sqrt(jnp.sum(a * a, axis=1, keepdims=True))
    return a / jnp.maximum(n, 1e-12)


def _full(shape):
    return pl.BlockSpec(shape, lambda i: (0,) * len(shape))


def _rows(shape):
    return pl.BlockSpec(shape, lambda i: (i,) + (0,) * (len(shape) - 1))


def _t1_body(x_ref, w1, b1, w2, b2, w3, b3, wz, bz, wd1, bd1, wd2, bd2,
             wd3, bd3, wx, bx, g0p, clusT,
             h1_o, h2_o, h3_o, z_o, q_o, xbar_o, s1_o):
    x = x_ref[...]
    h1 = _relu(_dot(x, w1[...]) + b1[...])
    h2 = _relu(_dot(h1, w2[...]) + b2[...])
    h3 = _relu(_dot(h2, w3[...]) + b3[...])
    z = _dot(h3, wz[...]) + bz[...]
    d1 = _relu(_dot(z, wd1[...]) + bd1[...])
    d2 = _relu(_dot(d1, wd2[...]) + bd2[...])
    d3 = _relu(_dot(d2, wd3[...]) + bd3[...])
    xbar_o[...] = _dot(d3, wx[...]) + bx[...]

    ct = clusT[...]
    z2s = jnp.sum(z * z, axis=1, keepdims=True)
    c2s = jnp.sum(ct * ct, axis=0)[None, :]
    dist = z2s + c2s - 2.0 * _dot(z, ct)
    q = 1.0 / (1.0 + dist / _V)
    q_o[...] = q / jnp.sum(q, axis=1, keepdims=True)

    s1 = _dot(x, g0p[...])
    for c in range(4):
        s1_o[c] = s1[:, c * 128:(c + 1) * 128]

    h1_o[...] = h1
    h2_o[...] = h2
    h3_o[...] = h3
    z_o[...] = z


def _stage_t1(x, p, clusT, g0p):
    B = 400
    G = _N // B
    outs = [
        jax.ShapeDtypeStruct((_N, 500), jnp.float32),
        jax.ShapeDtypeStruct((_N, 500), jnp.float32),
        jax.ShapeDtypeStruct((_N, 2000), jnp.float32),
        jax.ShapeDtypeStruct((_N, _NZ), jnp.float32),
        jax.ShapeDtypeStruct((_N, _NC), jnp.float32),
        jax.ShapeDtypeStruct((_N, _D), jnp.float32),
        jax.ShapeDtypeStruct((4, _N, 128), jnp.float32),
    ]
    args = [x]
    in_specs = [_rows((B, _D))]
    for name in ('enc1', 'enc2', 'enc3', 'zl', 'dec1', 'dec2', 'dec3', 'xbar'):
        w, b = p[name]
        args += [w, b.reshape(1, -1)]
        in_specs += [_full(w.shape), _full((1, b.shape[0]))]
    args += [g0p, clusT]
    in_specs += [_full(g0p.shape), _full(clusT.shape)]
    out_specs = [
        _rows((B, 500)), _rows((B, 500)), _rows((B, 2000)),
        _rows((B, _NZ)), _rows((B, _NC)), _rows((B, _D)),
        pl.BlockSpec((4, B, 128), lambda i: (0, i, 0)),
    ]
    return pl.pallas_call(
        _t1_body, grid=(G,), in_specs=in_specs, out_specs=out_specs,
        out_shape=outs)(*args)


def _make_att(Fh, Cin, Win, Freal, Cout, Wout, B=_BLK):
    def body(h_ref, zc_ref, mw, mb, gw, s_o, z_o):
        h = h_ref[...]
        zraw = jnp.concatenate([zc_ref[c] for c in range(Cin)], axis=1)
        zk = _relu(zraw[:, :Freal])
        mwv = mw[...]
        t = _dot(h, mwv[:Fh]) + _dot(zk, mwv[Fh:]) + mb[...]
        pcol = _norm2(_softmax(_leaky(t)))
        comb = pcol[:, 0:1] * zk + pcol[:, 1:2] * h
        s = _dot(comb, gw[...])
        for c in range(Cout):
            s_o[c] = s[:, c * Wout:(c + 1) * Wout]
        z_o[...] = zk

    def run(h, zc, mW, mB, gW):
        outs = [
            jax.ShapeDtypeStruct((Cout, _N, Wout), jnp.float32),
            jax.ShapeDtypeStruct((_N, Freal), jnp.float32),
        ]
        in_specs = [
            _rows((B, Fh)),
            pl.BlockSpec((Cin, B, Win), lambda i: (0, i, 0)),
            _full(mW.shape), _full((1, mB.shape[0])), _full(gW.shape),
        ]
        out_specs = [
            pl.BlockSpec((Cout, B, Wout), lambda i: (0, i, 0)),
            _rows((B, Freal)),
        ]
        return pl.pallas_call(
            body, grid=(_N // B,), in_specs=in_specs, out_specs=out_specs,
            out_shape=outs)(h, zc, mW, mB.reshape(1, -1), gW)
    return run


def _a4_body(z1_ref, z2_ref, z3_ref, z4c_ref, z_ref, mlw, mlb, gzp, s5_o):
    z1 = z1_ref[...]
    z2 = z2_ref[...]
    z3 = z3_ref[...]
    z4 = _relu(z4c_ref[0][:, :_NZ])
    zz = z_ref[...]
    mw = mlw[...]
    t = (_dot(z1, mw[0:500]) + _dot(z2, mw[500:1000]) + _dot(z3, mw[1000:3000])
         + _dot(z4, mw[3000:3010]) + _dot(zz, mw[3010:3020]) + mlb[...])
    w = _norm2(_softmax(_leaky(t)))
    g = gzp[...]
    s5 = (_dot(w[:, 0:1] * z1, g[0:500]) + _dot(w[:, 1:2] * z2, g[500:1000])
          + _dot(w[:, 2:3] * z3, g[1000:3000]) + _dot(w[:, 3:4] * z4, g[3000:3010])
          + _dot(w[:, 4:5] * zz, g[3010:3020]))
    s5_o[0] = s5


def _stage_a4(z1, z2, z3, z4c, z, mLW, mLB, gzp):
    in_specs = [
        _rows((_BLK, 500)), _rows((_BLK, 500)), _rows((_BLK, 2000)),
        pl.BlockSpec((1, _BLK, 128), lambda i: (0, i, 0)),
        _rows((_BLK, _NZ)),
        _full(mLW.shape), _full((1, 5)), _full(gzp.shape),
    ]
    out_specs = [pl.BlockSpec((1, _BLK, 128), lambda i: (0, i, 0))]
    outs = [jax.ShapeDtypeStruct((1, _N, 128), jnp.float32)]
    return pl.pallas_call(
        _a4_body, grid=(_GRID,), in_specs=in_specs, out_specs=out_specs,
        out_shape=outs)(z1, z2, z3, z4c, z, mLW, mLB.reshape(1, -1), gzp)[0]


def _a5_body(noc_ref, q_ref, mqw, mqb, pred_o, net_o, zf_o, loss_o):
    i = pl.program_id(0)
    no = noc_ref[0][:, :_NC]
    pred = _softmax(no)
    q = q_ref[...]
    mw = mqw[...]
    t = _dot(pred, mw[:_NC]) + _dot(q, mw[_NC:]) + mqb[...]
    pzh = _norm2(_softmax(_leaky(t)))
    zf = _softmax(pzh[:, 0:1] * pred + pzh[:, 1:2] * q)
    m = jnp.max(zf, axis=1, keepdims=True)
    col = lax.broadcasted_iota(jnp.int32, zf.shape, 1).astype(jnp.float32)
    amax = jnp.min(jnp.where(zf == m, col, 1e9), axis=1, keepdims=True)
    onehot = jnp.where(col == amax, 1.0, 0.0)
    wl = (_norm2(zf) >= 0.8).astype(jnp.float32)
    eps = 1e-12
    ln = wl * (onehot * jnp.log(zf + eps)
               + (1.0 - onehot) * jnp.log(1.0 - zf + eps))
    partial = -jnp.sum(ln) / (_N * _NC)

    @pl.when(i == 0)
    def _():
        loss_o[...] = jnp.zeros((1, 1), jnp.float32)

    loss_o[...] += jnp.full((1, 1), partial, jnp.float32)
    pred_o[...] = pred
    net_o[...] = no
    zf_o[...] = zf


def _stage_a5(noc, q, mQW, mQB):
    in_specs = [
        pl.BlockSpec((1, _BLK, 128), lambda i: (0, i, 0)),
        _rows((_BLK, _NC)),
        _full(mQW.shape), _full((1, 2)),
    ]
    out_specs = [
        _rows((_BLK, _NC)), _rows((_BLK, _NC)), _rows((_BLK, _NC)),
        pl.BlockSpec((1, 1), lambda i: (0, 0)),
    ]
    outs = [
        jax.ShapeDtypeStruct((_N, _NC), jnp.float32),
        jax.ShapeDtypeStruct((_N, _NC), jnp.float32),
        jax.ShapeDtypeStruct((_N, _NC), jnp.float32),
        jax.ShapeDtypeStruct((1, 1), jnp.float32),
    ]
    return pl.pallas_call(
        _a5_body, grid=(_GRID,), in_specs=in_specs, out_specs=out_specs,
        out_shape=outs)(noc, q, mQW, mQB.reshape(1, -1))


@functools.lru_cache(maxsize=None)
def _make_segsum(C, Wc):
    NCI = (C + 1) // 2
    mesh = plsc.VectorSubcoreMesh(core_axis_name="c", subcore_axis_name="s",
                                  num_cores=2, num_subcores=_TILES)

    @functools.partial(
        pl.kernel,
        out_type=jax.ShapeDtypeStruct((C * _N, Wc), jnp.float32),
        mesh=mesh,
        scratch_types=[
            [pltpu.VMEM((_K,), jnp.int32)] * 2,
            [pltpu.VMEM((_K,), jnp.int32)] * 2,
            [pltpu.VMEM((17 * _K,), jnp.float32)] * 2,
            [pltpu.VMEM((_K, Wc), jnp.float32)] * 2,
            pltpu.VMEM((16, Wc), jnp.float32),
            pltpu.VMEM_SHARED((_N, Wc), jnp.float32),
            [pltpu.SemaphoreType.DMA] * 2,
            [pltpu.SemaphoreType.DMA] * 2,
            [pltpu.SemaphoreType.DMA] * 2,
            pltpu.SemaphoreType.DMA,
        ],
    )
    def seg(s_hbm, srcadj_hbm, ewd_hbm, out_hbm,
            srca, dscat, ewd, rows, zbuf, acc,
            sem_i, sem_g, sem_s, sem_z):
        core = lax.axis_index("c")
        tid = lax.axis_index("s")

        def zrow(i, carry):
            for v in range(Wc // 16):
                zbuf[i, pl.ds(v * 16, 16)] = jnp.zeros((16,), jnp.float32)
            return carry
        lax.fori_loop(0, 16, zrow, 0)
        last = tid == _TILES - 1
        zstart = tid * _RA

        def load_idx(c, bi, slot):
            off = pl.multiple_of(tid * _EPT + bi * _K, 8)
            aoff = pl.multiple_of(c * _E_PAD + off, 8)
            eoff = pl.multiple_of((tid * _NB + bi) * (17 * _K), 8)
            pltpu.async_copy(srcadj_hbm.at[pl.ds(aoff, _K)], srca[slot],
                             sem_i[slot])
            pltpu.async_copy(ewd_hbm.at[pl.ds(eoff, 17 * _K)], ewd[slot],
                             sem_i[slot])

        def wait_idx(slot):
            pltpu.make_async_copy(srcadj_hbm.at[pl.ds(0, _K)], srca[slot],
                                  sem_i[slot]).wait()
            pltpu.make_async_copy(ewd_hbm.at[pl.ds(0, 17 * _K)], ewd[slot],
                                  sem_i[slot]).wait()

        def issue_gather(slot):
            pltpu.async_copy(s_hbm.at[srca[slot]], rows[slot], sem_g[slot])

        def wait_gather(slot):
            pltpu.make_async_copy(s_hbm.at[srca[slot]], rows[slot],
                                  sem_g[slot]).wait()

        def issue_scatter(slot):
            pltpu.async_copy(rows[slot], acc.at[dscat[slot]], sem_s[slot],
                             add=True)

        def wait_scatter(slot):
            pltpu.make_async_copy(rows[slot], acc.at[dscat[slot]],
                                  sem_s[slot]).wait()

        for ci in range(NCI):
            c = ci * 2 + core
            valid = c < C

            @pl.when(valid)
            def _():
                zds = []
                for i in range(_RB // 16):
                    zds.append(pltpu.async_copy(
                        zbuf, acc.at[pl.ds(zstart + i * 16, 16)], sem_z))
                for d in zds:
                    d.wait()

            plsc.subcore_barrier()

            @pl.when(valid)
            def _():
                load_idx(c, 0, 0)
                wait_idx(0)
                issue_gather(0)
                load_idx(c, 1, 1)

                def body2(i, carry):
                    for ph in (0, 1):
                        b = i * 2 + ph
                        p, q = ph, 1 - ph
                        wait_idx(q)

                        @pl.when(b >= 1)
                        def _():
                            wait_scatter(q)

                        issue_gather(q)
                        wait_gather(p)

                        def scale(i4, c2):
                            e0 = i4 * 4
                            for u in range(4):
                                e = e0 + u
                                w = ewd[p][pl.ds(e * 16, 16)]
                                for v in range(Wc // 16):
                                    sl = pl.ds(v * 16, 16)
                                    rows[p][e, sl] = rows[p][e, sl] * w
                            return c2
                        lax.fori_loop(0, _K // 4, scale, 0)
                        for v in range(_K // 16):
                            sl = pl.ds(v * 16, 16)
                            dscat[p][sl] = ewd[p][pl.ds(
                                16 * _K + v * 16, 16)].astype(jnp.int32)
                        load_idx(c, jnp.minimum(b + 2, _NB - 1), p)
                        issue_scatter(p)
                    return carry
                lax.fori_loop(0, _NB // 2, body2, 0)
                wait_gather(0)
                wait_idx(1)
                wait_scatter(1)

            plsc.subcore_barrier()

            @pl.when(valid & jnp.logical_not(last))
            def _():
                pltpu.sync_copy(
                    acc.at[pl.ds(tid * _RA, _RA)],
                    out_hbm.at[pl.ds(c * _N + tid * _RA, _RA)])

            @pl.when(valid & last)
            def _():
                pltpu.sync_copy(
                    acc.at[pl.ds(15 * _RA, _RB)],
                    out_hbm.at[pl.ds(c * _N + 15 * _RA, _RB)])

            plsc.subcore_barrier()

    return seg


_att1 = _make_att(500, 4, 128, 500, 4, 128)
_att2 = _make_att(500, 4, 128, 500, 16, 128)
_att3 = _make_att(2000, 16, 128, 2000, 1, 128, B=400)


def _padcols(w, cols):
    return jnp.pad(w, ((0, 0), (0, cols - w.shape[1])))


def kernel(x, edge_index, edge_weight, params):
    p = params
    src = edge_index[0].astype(jnp.int32)
    dst = edge_index[1].astype(jnp.int32)
    pad = _E_PAD - _E
    src_p = jnp.pad(src, (0, pad))
    dst_p = jnp.pad(dst, (0, pad))
    ew_p = jnp.pad(edge_weight.astype(jnp.float32), (0, pad))
    ewx = jnp.broadcast_to(ew_p[:, None], (_E_PAD, 16))
    ewd = jnp.concatenate(
        [ewx.reshape(_TILES, _NB, _K * 16),
         dst_p.astype(jnp.float32).reshape(_TILES, _NB, _K)],
        axis=2).reshape(-1)

    offs4 = (jnp.arange(4, dtype=jnp.int32) * _N)[:, None]
    srcadj4 = (offs4 + src_p[None, :]).reshape(-1)
    offs16 = (jnp.arange(16, dtype=jnp.int32) * _N)[:, None]
    srcadj16 = (offs16 + src_p[None, :]).reshape(-1)

    g0p = _padcols(p['g0'], 512)
    g1p = _padcols(p['g1'], 512)
    g2p = _padcols(p['g2'], 2048)
    g3p = _padcols(p['g3'], 128)
    gzp = _padcols(p['gz'], 128)
    clusT = p['cluster'].T

    h1, h2, h3, z, q, x_bar, s1c = _stage_t1(x, p, clusT, g0p)

    seg1 = _make_segsum(4, 128)(s1c.reshape(4 * _N, 128), srcadj4, ewd)
    s2c, z1 = _att1(h1, seg1.reshape(4, _N, 128), p['m1'][0], p['m1'][1], g1p)

    seg2 = _make_segsum(4, 128)(s2c.reshape(4 * _N, 128), srcadj4, ewd)
    s3c, z2 = _att2(h2, seg2.reshape(4, _N, 128), p['m2'][0], p['m2'][1], g2p)

    seg3 = _make_segsum(16, 128)(s3c.reshape(16 * _N, 128), srcadj16, ewd)
    s4c, z3 = _att3(h3, seg3.reshape(16, _N, 128), p['m3'][0], p['m3'][1], g3p)

    seg4 = _make_segsum(1, 128)(s4c.reshape(_N, 128), src_p, ewd)
    s5c = _stage_a4(z1, z2, z3, seg4.reshape(1, _N, 128), z,
                    p['mL'][0], p['mL'][1], gzp)

    seg5 = _make_segsum(1, 128)(s5c.reshape(_N, 128), src_p, ewd)
    predict, net_output, z_F, loss = _stage_a5(
        seg5.reshape(1, _N, 128), q, p['mZQ'][0], p['mZQ'][1])

    return (x_bar, q, predict, z, net_output, loss[0, 0], z_F)
